# baseline (speedup 1.0000x reference)
import numpy as np

import jax
import jax.numpy as jnp
from jax import lax
from jax.experimental import pallas as pl
from jax.experimental.pallas import tpu as pltpu
from jax.experimental.pallas import tpu_sc as plsc

N = 10000
E = 320000
D = 128

NUM_TILES = 16
LANES = 16
SUB = 80
NG = 252
EP_TILE = NG * SUB
EP = EP_TILE * NUM_TILES

ROWS_PER_TILE = 640
LAST_ROWS = N - 15 * ROWS_PER_TILE

_Q = np.zeros(D, np.int32)
for _j in range(4):
    for _l in range(16):
        _Q[32 * _j + 2 * _l] = 32 * _j + _l
        _Q[32 * _j + 2 * _l + 1] = 32 * _j + 16 + _l


def _mm_body(x_ref, wi_ref, wo_ref, h_ref):
    x = x_ref[...]
    h_ref[0] = jnp.dot(x, wi_ref[...],
                       preferred_element_type=jnp.float32).astype(jnp.bfloat16)
    h_ref[1] = jnp.dot(x, wo_ref[...],
                       preferred_element_type=jnp.float32).astype(jnp.bfloat16)


def _combine_body(acc_ref, bin_ref, bout_ref, cin_ref, cout_ref, o_ref):
    o_ref[...] = (cin_ref[...] * (acc_ref[0] + bin_ref[...])
                  + cout_ref[...] * (acc_ref[1] + bout_ref[...]))


def _prop_body(h_hbm, edata_hbm, acc_hbm,
               e0, e1, e2, e3, r0, r1, r2, r3, s0, s1, d0, d1,
               semE, semG, semS, acc_sh):
    c = lax.axis_index("c")
    s = lax.axis_index("s")
    ebufs = [e0, e1, e2, e3]
    rbufs = [r0, r1, r2, r3]
    sbufs = [s0, s1]
    dbufs = [d0, d1]

    zero16 = jnp.zeros((LANES,), jnp.float32)

    def zrow(i, _):
        for j in range(D // LANES):
            s0[i, pl.ds(j * LANES, LANES)] = zero16
        return 0

    lax.fori_loop(0, SUB, zrow, 0)

    row0 = s * ROWS_PER_TILE

    @pl.when(s < 15)
    def _():
        for k in range(ROWS_PER_TILE // SUB):
            pltpu.sync_copy(s0.at[pl.ds(0, SUB)],
                            acc_sh.at[pl.ds(row0 + k * SUB, SUB)])

    @pl.when(s == 15)
    def _():
        for k in range(LAST_ROWS // SUB):
            pltpu.sync_copy(s0.at[pl.ds(0, SUB)],
                            acc_sh.at[pl.ds(row0 + k * SUB, SUB)])

    plsc.subcore_barrier()

    hc = h_hbm.at[c]
    g0 = s * NG

    def ecopy_start(g, es):
        pltpu.async_copy(edata_hbm.at[c, g0 + g], ebufs[es], semE.at[es])

    def ecopy_wait(g, es):
        pltpu.make_async_copy(edata_hbm.at[c, g0 + g], ebufs[es],
                              semE.at[es]).wait()

    def gather_start(es):
        pltpu.async_copy(hc.at[ebufs[es].at[0]], rbufs[es], semG.at[es])

    def gather_wait(es):
        pltpu.make_async_copy(hc.at[ebufs[es].at[0]], rbufs[es],
                              semG.at[es]).wait()

    def scatter_start(ss):
        pltpu.async_copy(sbufs[ss], acc_sh.at[dbufs[ss]],
                         semS.at[ss], add=True)

    def scatter_wait(ss):
        pltpu.make_async_copy(sbufs[ss], acc_sh.at[dbufs[ss]],
                              semS.at[ss]).wait()

    def scale(es, ss):
        eb, rb, sb, db = ebufs[es], rbufs[es], sbufs[ss], dbufs[ss]
        for t in range(SUB // LANES):
            db[pl.ds(t * LANES, LANES)] = eb[1, pl.ds(t * LANES, LANES)]

        def sblk(t, _):
            w16 = lax.bitcast_convert_type(eb[2, pl.ds(t * LANES, LANES)],
                                           jnp.float32)
            rbase = t * LANES
            for l in range(LANES):
                w = w16[l]
                e = rbase + l
                for k in range(4):
                    chunk = rb[e, pl.ds(LANES * k, LANES)]
                    f_lo = lax.bitcast_convert_type(
                        jnp.left_shift(chunk, 16), jnp.float32)
                    f_hi = lax.bitcast_convert_type(
                        jnp.bitwise_and(chunk, jnp.int32(-65536)), jnp.float32)
                    sb[e, pl.ds(32 * k, LANES)] = f_lo * w
                    sb[e, pl.ds(32 * k + LANES, LANES)] = f_hi * w
            return 0

        lax.fori_loop(0, SUB // LANES, sblk, 0)

    ecopy_start(0, 0)
    ecopy_start(1, 1)
    ecopy_start(2, 2)
    ecopy_wait(0, 0)
    gather_start(0)
    ecopy_wait(1, 1)
    gather_start(1)

    def quad_body(q, _):
        gq = q * 4
        for slot in range(4):
            g = gq + slot
            @pl.when(g >= 2)
            def _():
                scatter_wait(slot % 2)

            @pl.when(g + 3 < NG)
            def _():
                ecopy_start(g + 3, (slot + 3) % 4)

            @pl.when(g + 2 < NG)
            def _():
                ecopy_wait(g + 2, (slot + 2) % 4)
                gather_start((slot + 2) % 4)

            gather_wait(slot)
            scale(slot, slot % 2)
            scatter_start(slot % 2)
        return 0

    lax.fori_loop(0, NG // 4, quad_body, 0)
    scatter_wait(0)
    scatter_wait(1)

    plsc.subcore_barrier()

    @pl.when(s < 15)
    def _():
        pltpu.sync_copy(acc_sh.at[pl.ds(row0, ROWS_PER_TILE)],
                        acc_hbm.at[c, pl.ds(row0, ROWS_PER_TILE)])

    @pl.when(s == 15)
    def _():
        pltpu.sync_copy(acc_sh.at[pl.ds(row0, LAST_ROWS)],
                        acc_hbm.at[c, pl.ds(row0, LAST_ROWS)])


@jax.jit
def kernel(x, edge_index_in, edge_weight_in, edge_index_out, edge_weight_out,
           W_main_in, W_main_out, W_shared,
           b_main_in, b_main_out, b_shared_in, b_shared_out,
           C_in_vec, C_out_vec):
    q = jnp.asarray(_Q)
    w_in = (W_main_in + W_shared)[:, q]
    w_out = (W_main_out + W_shared)[:, q]
    h = pl.pallas_call(
        _mm_body,
        out_shape=jax.ShapeDtypeStruct((2, N, D), jnp.bfloat16),
    )(x, w_in, w_out)
    hbits = lax.bitcast_convert_type(h.reshape(2, N, D // 2, 2), jnp.int32)

    pad = EP - E

    def prep(idx, w):
        src = jnp.concatenate([idx[0], jnp.zeros((pad,), jnp.int32)])
        dst = jnp.concatenate([idx[1], jnp.zeros((pad,), jnp.int32)])
        wb = jnp.concatenate([w, jnp.zeros((pad,), jnp.float32)])
        wi = lax.bitcast_convert_type(wb, jnp.int32)
        rec = jnp.stack([src, dst, wi])
        return rec.reshape(3, EP // SUB, SUB).transpose(1, 0, 2)

    edata = jnp.stack([prep(edge_index_in, edge_weight_in),
                       prep(edge_index_out, edge_weight_out)])

    prop = pl.kernel(
        _prop_body,
        out_type=jax.ShapeDtypeStruct((2, N, D), jnp.float32),
        mesh=plsc.VectorSubcoreMesh(core_axis_name="c", subcore_axis_name="s"),
        compiler_params=pltpu.CompilerParams(use_tc_tiling_on_sc=False),
        scratch_types=(
            [pltpu.VMEM((3, SUB), jnp.int32) for _ in range(4)]
            + [pltpu.VMEM((SUB, D // 2), jnp.int32) for _ in range(4)]
            + [pltpu.VMEM((SUB, D), jnp.float32) for _ in range(2)]
            + [pltpu.VMEM((SUB,), jnp.int32) for _ in range(2)]
            + [pltpu.SemaphoreType.DMA((4,)),
               pltpu.SemaphoreType.DMA((4,)),
               pltpu.SemaphoreType.DMA((2,))]
            + [pltpu.VMEM_SHARED((N, D), jnp.float32)]
        ),
    )
    acc = prop(hbits, edata)

    b_in = (b_main_in + b_shared_in).reshape(1, D)
    b_out = (b_main_out + b_shared_out).reshape(1, D)
    out = pl.pallas_call(
        _combine_body,
        out_shape=jax.ShapeDtypeStruct((N, D), jnp.float32),
    )(acc, b_in, b_out, C_in_vec, C_out_vec)
    return out


# speedup vs baseline: 1.5131x; 1.5131x over previous
import numpy as np

import jax
import jax.numpy as jnp
from jax import lax
from jax.experimental import pallas as pl
from jax.experimental.pallas import tpu as pltpu
from jax.experimental.pallas import tpu_sc as plsc

N = 10000
E = 320000
D = 128

NUM_TILES = 16
LANES = 16
SUB = 80
NG = 252
EP_TILE = NG * SUB
EP = EP_TILE * NUM_TILES

ROWS_PER_TILE = 640
LAST_ROWS = N - 15 * ROWS_PER_TILE

_Q = np.zeros(D, np.int32)
for _j in range(4):
    for _l in range(16):
        _Q[32 * _j + 2 * _l] = 32 * _j + _l
        _Q[32 * _j + 2 * _l + 1] = 32 * _j + 16 + _l


def _mm_body(x_ref, wi_ref, wo_ref, h_ref):
    x = x_ref[...]
    h_ref[0] = jnp.dot(x, wi_ref[...], preferred_element_type=jnp.float32)
    h_ref[1] = jnp.dot(x, wo_ref[...], preferred_element_type=jnp.float32)


def _combine_body(acc_ref, bin_ref, bout_ref, cin_ref, cout_ref, o_ref):
    o_ref[...] = (cin_ref[...] * (acc_ref[0] + bin_ref[...])
                  + cout_ref[...] * (acc_ref[1] + bout_ref[...]))


def _prop_body(h_hbm, edata_hbm, acc_hbm,
               e0, e1, e2, e3, r0, r1, r2, r3, d0, d1,
               semE, semG, semS, acc_sh):
    c = lax.axis_index("c")
    s = lax.axis_index("s")
    ebufs = [e0, e1, e2, e3]
    rbufs = [r0, r1, r2, r3]
    dbufs = [d0, d1]

    zero16 = jnp.zeros((LANES,), jnp.float32)

    def zrow(i, _):
        for j in range(D // LANES):
            r0[i, pl.ds(j * LANES, LANES)] = zero16
        return 0

    lax.fori_loop(0, SUB, zrow, 0)

    row0 = s * ROWS_PER_TILE

    @pl.when(s < 15)
    def _():
        for k in range(ROWS_PER_TILE // SUB):
            pltpu.sync_copy(r0.at[pl.ds(0, SUB)],
                            acc_sh.at[pl.ds(row0 + k * SUB, SUB)])

    @pl.when(s == 15)
    def _():
        for k in range(LAST_ROWS // SUB):
            pltpu.sync_copy(r0.at[pl.ds(0, SUB)],
                            acc_sh.at[pl.ds(row0 + k * SUB, SUB)])

    plsc.subcore_barrier()

    hc = h_hbm.at[c]
    g0 = s * NG

    def ecopy_start(g, es):
        pltpu.async_copy(edata_hbm.at[c, g0 + g], ebufs[es], semE.at[es])

    def ecopy_wait(g, es):
        pltpu.make_async_copy(edata_hbm.at[c, g0 + g], ebufs[es],
                              semE.at[es]).wait()

    HALF = SUB // 2

    def gather_start(es):
        pltpu.async_copy(hc.at[ebufs[es].at[0, pl.ds(0, HALF)]],
                         rbufs[es].at[pl.ds(0, HALF)], semG.at[es])
        pltpu.async_copy(hc.at[ebufs[es].at[0, pl.ds(HALF, HALF)]],
                         rbufs[es].at[pl.ds(HALF, HALF)], semG.at[es])

    def gather_wait(es):
        pltpu.make_async_copy(hc.at[ebufs[es].at[0, pl.ds(0, HALF)]],
                              rbufs[es].at[pl.ds(0, HALF)], semG.at[es]).wait()
        pltpu.make_async_copy(hc.at[ebufs[es].at[0, pl.ds(HALF, HALF)]],
                              rbufs[es].at[pl.ds(HALF, HALF)],
                              semG.at[es]).wait()

    def scatter_start(es, ss):
        pltpu.async_copy(rbufs[es], acc_sh.at[dbufs[ss]],
                         semS.at[ss], add=True)

    def scatter_wait(es, ss):
        pltpu.make_async_copy(rbufs[es], acc_sh.at[dbufs[ss]],
                              semS.at[ss]).wait()

    def scale(es, ss):
        eb, rb, db = ebufs[es], rbufs[es], dbufs[ss]
        for t in range(SUB // LANES):
            db[pl.ds(t * LANES, LANES)] = eb[1, pl.ds(t * LANES, LANES)]

        def sblk(t, _):
            w16 = lax.bitcast_convert_type(eb[2, pl.ds(t * LANES, LANES)],
                                           jnp.float32)
            rbase = t * LANES
            for l in range(LANES):
                w = w16[l]
                e = rbase + l
                for k in range(D // LANES):
                    sl = pl.ds(k * LANES, LANES)
                    rb[e, sl] = rb[e, sl] * w
            return 0

        lax.fori_loop(0, SUB // LANES, sblk, 0)

    ecopy_start(0, 0)
    ecopy_start(1, 1)
    ecopy_start(2, 2)
    ecopy_wait(0, 0)
    gather_start(0)
    ecopy_wait(1, 1)
    gather_start(1)

    def quad_body(q, _):
        gq = q * 4
        for slot in range(4):
            g = gq + slot
            @pl.when(g >= 2)
            def _():
                scatter_wait((slot + 2) % 4, slot % 2)

            @pl.when(g + 3 < NG)
            def _():
                ecopy_start(g + 3, (slot + 3) % 4)

            @pl.when(g + 2 < NG)
            def _():
                ecopy_wait(g + 2, (slot + 2) % 4)
                gather_start((slot + 2) % 4)

            gather_wait(slot)
            scale(slot, slot % 2)
            scatter_start(slot, slot % 2)
        return 0

    lax.fori_loop(0, NG // 4, quad_body, 0)
    scatter_wait(2, 0)
    scatter_wait(3, 1)

    plsc.subcore_barrier()

    @pl.when(s < 15)
    def _():
        pltpu.sync_copy(acc_sh.at[pl.ds(row0, ROWS_PER_TILE)],
                        acc_hbm.at[c, pl.ds(row0, ROWS_PER_TILE)])

    @pl.when(s == 15)
    def _():
        pltpu.sync_copy(acc_sh.at[pl.ds(row0, LAST_ROWS)],
                        acc_hbm.at[c, pl.ds(row0, LAST_ROWS)])


@jax.jit
def kernel(x, edge_index_in, edge_weight_in, edge_index_out, edge_weight_out,
           W_main_in, W_main_out, W_shared,
           b_main_in, b_main_out, b_shared_in, b_shared_out,
           C_in_vec, C_out_vec):
    w_in = W_main_in + W_shared
    w_out = W_main_out + W_shared
    h = pl.pallas_call(
        _mm_body,
        out_shape=jax.ShapeDtypeStruct((2, N, D), jnp.float32),
    )(x, w_in, w_out)

    pad = EP - E

    def prep(idx, w):
        src = jnp.concatenate([idx[0], jnp.zeros((pad,), jnp.int32)])
        dst = jnp.concatenate([idx[1], jnp.zeros((pad,), jnp.int32)])
        wb = jnp.concatenate([w, jnp.zeros((pad,), jnp.float32)])
        wi = lax.bitcast_convert_type(wb, jnp.int32)
        rec = jnp.stack([src, dst, wi])
        return rec.reshape(3, EP // SUB, SUB).transpose(1, 0, 2)

    edata = jnp.stack([prep(edge_index_in, edge_weight_in),
                       prep(edge_index_out, edge_weight_out)])

    prop = pl.kernel(
        _prop_body,
        out_type=jax.ShapeDtypeStruct((2, N, D), jnp.float32),
        mesh=plsc.VectorSubcoreMesh(core_axis_name="c", subcore_axis_name="s"),
        scratch_types=(
            [pltpu.VMEM((3, SUB), jnp.int32) for _ in range(4)]
            + [pltpu.VMEM((SUB, D), jnp.float32) for _ in range(4)]
            + [pltpu.VMEM((SUB,), jnp.int32) for _ in range(2)]
            + [pltpu.SemaphoreType.DMA((4,)),
               pltpu.SemaphoreType.DMA((4,)),
               pltpu.SemaphoreType.DMA((2,))]
            + [pltpu.VMEM_SHARED((N, D), jnp.float32)]
        ),
    )
    acc = prop(h, edata)

    b_in = (b_main_in + b_shared_in).reshape(1, D)
    b_out = (b_main_out + b_shared_out).reshape(1, D)
    out = pl.pallas_call(
        _combine_body,
        out_shape=jax.ShapeDtypeStruct((N, D), jnp.float32),
    )(acc, b_in, b_out, C_in_vec, C_out_vec)
    return out


# speedup vs baseline: 1.8148x; 1.1994x over previous
import numpy as np

import jax
import jax.numpy as jnp
from jax import lax
from jax.experimental import pallas as pl
from jax.experimental.pallas import tpu as pltpu
from jax.experimental.pallas import tpu_sc as plsc

N = 10000
E = 320000
D = 128

NUM_TILES = 16
LANES = 16
SUB = 88
NG = 228
EP_TILE = NG * SUB
EP = EP_TILE * NUM_TILES

ROWS_PER_TILE = 640
LAST_ROWS = N - 15 * ROWS_PER_TILE

_Q = np.zeros(D, np.int32)
for _j in range(4):
    for _l in range(16):
        _Q[32 * _j + 2 * _l] = 32 * _j + _l
        _Q[32 * _j + 2 * _l + 1] = 32 * _j + 16 + _l


def _mm_body(x_ref, wi_ref, wo_ref, h_ref):
    x = x_ref[...]
    h_ref[0] = jnp.dot(x, wi_ref[...], preferred_element_type=jnp.float32)
    h_ref[1] = jnp.dot(x, wo_ref[...], preferred_element_type=jnp.float32)


def _combine_body(acc_ref, bin_ref, bout_ref, cin_ref, cout_ref, o_ref):
    o_ref[...] = (cin_ref[...] * (acc_ref[0] + bin_ref[...])
                  + cout_ref[...] * (acc_ref[1] + bout_ref[...]))


def _prop_body(h_hbm, edata_hbm, acc_hbm,
               e0, e1, e2, e3, r0, r1, r2, r3, d0, d1,
               semE, semG, semS, acc_sh):
    c = lax.axis_index("c")
    s = lax.axis_index("s")
    ebufs = [e0, e1, e2, e3]
    rbufs = [r0, r1, r2, r3]
    dbufs = [d0, d1]

    zero16 = jnp.zeros((LANES,), jnp.float32)

    def zrow(i, _):
        for j in range(D // LANES):
            r0[i, pl.ds(j * LANES, LANES)] = zero16
        return 0

    lax.fori_loop(0, SUB, zrow, 0)

    row0 = s * ROWS_PER_TILE

    @pl.when(s < 15)
    def _():
        for k in range(ROWS_PER_TILE // SUB):
            pltpu.sync_copy(r0.at[pl.ds(0, SUB)],
                            acc_sh.at[pl.ds(row0 + k * SUB, SUB)])
        rem = ROWS_PER_TILE % SUB
        if rem:
            pltpu.sync_copy(r0.at[pl.ds(0, rem)],
                            acc_sh.at[pl.ds(row0 + ROWS_PER_TILE - rem, rem)])

    @pl.when(s == 15)
    def _():
        for k in range(LAST_ROWS // SUB):
            pltpu.sync_copy(r0.at[pl.ds(0, SUB)],
                            acc_sh.at[pl.ds(row0 + k * SUB, SUB)])
        rem = LAST_ROWS % SUB
        if rem:
            pltpu.sync_copy(r0.at[pl.ds(0, rem)],
                            acc_sh.at[pl.ds(row0 + LAST_ROWS - rem, rem)])

    plsc.subcore_barrier()

    hc = h_hbm.at[c]
    g0 = s * NG

    def ecopy_start(g, es):
        pltpu.async_copy(edata_hbm.at[c, g0 + g], ebufs[es], semE.at[es])

    def ecopy_wait(g, es):
        pltpu.make_async_copy(edata_hbm.at[c, g0 + g], ebufs[es],
                              semE.at[es]).wait()

    def gather_start(es):
        pltpu.async_copy(hc.at[ebufs[es].at[0]], rbufs[es], semG.at[es])

    def gather_wait(es):
        pltpu.make_async_copy(hc.at[ebufs[es].at[0]], rbufs[es],
                              semG.at[es]).wait()

    def scatter_start(es, ss):
        pltpu.async_copy(rbufs[es], acc_sh.at[dbufs[ss]],
                         semS.at[ss], add=True)

    def scatter_wait(es, ss):
        pltpu.make_async_copy(rbufs[es], acc_sh.at[dbufs[ss]],
                              semS.at[ss]).wait()

    def scale(es, ss):
        eb, rb, db = ebufs[es], rbufs[es], dbufs[ss]
        for t in range(SUB // LANES):
            db[pl.ds(t * LANES, LANES)] = eb[1, pl.ds(t * LANES, LANES)]

        def sblk(t, _):
            w16 = lax.bitcast_convert_type(eb[2, pl.ds(t * LANES, LANES)],
                                           jnp.float32)
            rbase = t * LANES
            for l in range(LANES):
                w = w16[l]
                e = rbase + l
                for k in range(D // LANES):
                    sl = pl.ds(k * LANES, LANES)
                    rb[e, sl] = rb[e, sl] * w
            return 0

        lax.fori_loop(0, SUB // LANES, sblk, 0)

    ecopy_start(0, 0)
    ecopy_start(1, 1)
    ecopy_start(2, 2)
    ecopy_wait(0, 0)
    gather_start(0)
    ecopy_wait(1, 1)
    gather_start(1)

    def quad_body(q, _):
        gq = q * 4
        for slot in range(4):
            g = gq + slot
            @pl.when(g >= 2)
            def _():
                scatter_wait((slot + 2) % 4, slot % 2)

            @pl.when(g + 3 < NG)
            def _():
                ecopy_start(g + 3, (slot + 3) % 4)

            @pl.when(g + 2 < NG)
            def _():
                ecopy_wait(g + 2, (slot + 2) % 4)
                gather_start((slot + 2) % 4)

            gather_wait(slot)
            scale(slot, slot % 2)
            scatter_start(slot, slot % 2)
        return 0

    lax.fori_loop(0, NG // 4, quad_body, 0)
    scatter_wait(2, 0)
    scatter_wait(3, 1)

    plsc.subcore_barrier()

    @pl.when(s < 15)
    def _():
        pltpu.sync_copy(acc_sh.at[pl.ds(row0, ROWS_PER_TILE)],
                        acc_hbm.at[c, pl.ds(row0, ROWS_PER_TILE)])

    @pl.when(s == 15)
    def _():
        pltpu.sync_copy(acc_sh.at[pl.ds(row0, LAST_ROWS)],
                        acc_hbm.at[c, pl.ds(row0, LAST_ROWS)])


@jax.jit
def kernel(x, edge_index_in, edge_weight_in, edge_index_out, edge_weight_out,
           W_main_in, W_main_out, W_shared,
           b_main_in, b_main_out, b_shared_in, b_shared_out,
           C_in_vec, C_out_vec):
    w_in = W_main_in + W_shared
    w_out = W_main_out + W_shared
    h = pl.pallas_call(
        _mm_body,
        out_shape=jax.ShapeDtypeStruct((2, N, D), jnp.float32),
    )(x, w_in, w_out)

    pad = EP - E

    def prep(idx, w):
        src = jnp.concatenate([idx[0], jnp.zeros((pad,), jnp.int32)])
        dst = jnp.concatenate([idx[1], jnp.zeros((pad,), jnp.int32)])
        wb = jnp.concatenate([w, jnp.zeros((pad,), jnp.float32)])
        wi = lax.bitcast_convert_type(wb, jnp.int32)
        rec = jnp.stack([src, dst, wi])
        return rec.reshape(3, EP // SUB, SUB).transpose(1, 0, 2)

    edata = jnp.stack([prep(edge_index_in, edge_weight_in),
                       prep(edge_index_out, edge_weight_out)])

    prop = pl.kernel(
        _prop_body,
        out_type=jax.ShapeDtypeStruct((2, N, D), jnp.float32),
        mesh=plsc.VectorSubcoreMesh(core_axis_name="c", subcore_axis_name="s"),
        scratch_types=(
            [pltpu.VMEM((3, SUB), jnp.int32) for _ in range(4)]
            + [pltpu.VMEM((SUB, D), jnp.float32) for _ in range(4)]
            + [pltpu.VMEM((SUB,), jnp.int32) for _ in range(2)]
            + [pltpu.SemaphoreType.DMA((4,)),
               pltpu.SemaphoreType.DMA((4,)),
               pltpu.SemaphoreType.DMA((2,))]
            + [pltpu.VMEM_SHARED((N, D), jnp.float32)]
        ),
    )
    acc = prop(h, edata)

    b_in = (b_main_in + b_shared_in).reshape(1, D)
    b_out = (b_main_out + b_shared_out).reshape(1, D)
    out = pl.pallas_call(
        _combine_body,
        out_shape=jax.ShapeDtypeStruct((N, D), jnp.float32),
    )(acc, b_in, b_out, C_in_vec, C_out_vec)
    return out
